# baseline (speedup 1.0000x reference)
import functools

import jax
import jax.numpy as jnp
from jax import lax
from jax.experimental import pallas as pl
from jax.experimental.pallas import tpu as pltpu
from jax.experimental.pallas import tpu_sc as plsc


_SCALE = 32.0

_NC = 2
_NS = 16
_NW = _NC * _NS

_B = 4
_SEQ = 8192
_D = 1024
_FLAT = _SEQ * _D
_W = _FLAT // _NW
_CH = 16384
_NCHUNK = _W // _CH
_UNROLL = 8


def _fma_chunk(xbuf, pebuf):
    def body(i, carry):
        base = i * (16 * _UNROLL)
        for u in range(_UNROLL):
            sl = pl.ds(base + u * 16, 16)
            xbuf[sl] = xbuf[sl] * _SCALE + pebuf[sl]
        return carry
    lax.fori_loop(0, _CH // (16 * _UNROLL), body, 0)


def _sc_body(x_hbm, pe_hbm, out_hbm, xa, xb, pea, peb, sla, slb, ssa, ssb, spe):
    wid = lax.axis_index("s") * _NC + lax.axis_index("c")
    base = wid * _W
    xbufs = (xa, xb)
    lsems = (sla, slb)
    ssems = (ssa, ssb)
    pebufs = (pea, peb)

    pltpu.sync_copy(pe_hbm.at[pl.ds(base, _CH)], pea)

    n_iter = _NCHUNK * _B
    loads = {}
    pe_loads = {}
    stores = {}
    loads[0] = pltpu.async_copy(
        x_hbm.at[0, pl.ds(base, _CH)], xbufs[0], lsems[0])
    for t in range(n_iter):
        c, b = divmod(t, _B)
        cur = t % 2
        if t + 1 < n_iter:
            nxt = (t + 1) % 2
            if t - 1 >= 0:
                stores[t - 1].wait()
            c2, b2 = divmod(t + 1, _B)
            loads[t + 1] = pltpu.async_copy(
                x_hbm.at[b2, pl.ds(base + c2 * _CH, _CH)],
                xbufs[nxt], lsems[nxt])
        if b == 0 and c + 1 < _NCHUNK:
            pe_loads[c + 1] = pltpu.async_copy(
                pe_hbm.at[pl.ds(base + (c + 1) * _CH, _CH)],
                pebufs[(c + 1) % 2], spe)
        if b == 0 and c > 0:
            pe_loads[c].wait()
        loads[t].wait()
        _fma_chunk(xbufs[cur], pebufs[c % 2])
        stores[t] = pltpu.async_copy(
            xbufs[cur], out_hbm.at[b, pl.ds(base + c * _CH, _CH)], ssems[cur])
    stores[n_iter - 2].wait()
    stores[n_iter - 1].wait()


def _sc_call(xf, pef):
    mesh = plsc.VectorSubcoreMesh(core_axis_name="c", subcore_axis_name="s")
    run = functools.partial(
        pl.kernel,
        mesh=mesh,
        out_type=jax.ShapeDtypeStruct((_B, _FLAT), jnp.float32),
        scratch_types=[
            pltpu.VMEM((_CH,), jnp.float32),
            pltpu.VMEM((_CH,), jnp.float32),
            pltpu.VMEM((_CH,), jnp.float32),
            pltpu.VMEM((_CH,), jnp.float32),
            pltpu.SemaphoreType.DMA,
            pltpu.SemaphoreType.DMA,
            pltpu.SemaphoreType.DMA,
            pltpu.SemaphoreType.DMA,
            pltpu.SemaphoreType.DMA,
        ],
    )(_sc_body)
    return run(xf, pef)


def kernel(x, pos_emb):
    b, s, d = x.shape
    xf = x.reshape(b, s * d)
    pef = pos_emb[:s].reshape(s * d)
    out = _sc_call(xf, pef)
    return out.reshape(b, s, d)


# speedup vs baseline: 2.4396x; 2.4396x over previous
import functools

import jax
import jax.numpy as jnp
from jax import lax
from jax.experimental import pallas as pl
from jax.experimental.pallas import tpu as pltpu
from jax.experimental.pallas import tpu_sc as plsc


_SCALE = 32.0

_NC = 2
_NS = 16
_NW = _NC * _NS

_B = 4
_SEQ = 8192
_D = 1024
_ROWS_W = _SEQ // _NW
_R = 16
_NCHUNK = _ROWS_W // _R
_VPR = _D // 16
_UNROLL = 8


def _fma_chunk(xbuf, pebuf):
    def body(k, carry):
        i = k >> 3
        j = (k & 7) * (16 * _UNROLL)
        for u in range(_UNROLL):
            sl = pl.ds(j + u * 16, 16)
            xbuf[i, sl] = xbuf[i, sl] * _SCALE + pebuf[i, sl]
        return carry
    lax.fori_loop(0, _R * (_VPR // _UNROLL), body, 0)


def _sc_body(x_hbm, pe_hbm, out_hbm, xa, xb, pea, peb, sla, slb, ssa, ssb, spe):
    wid = lax.axis_index("s") * _NC + lax.axis_index("c")
    row0 = wid * _ROWS_W
    xbufs = (xa, xb)
    lsems = (sla, slb)
    ssems = (ssa, ssb)
    pebufs = (pea, peb)

    pltpu.sync_copy(pe_hbm.at[pl.ds(row0, _R), :], pea)

    n_iter = _NCHUNK * _B
    loads = {}
    pe_loads = {}
    stores = {}
    loads[0] = pltpu.async_copy(
        x_hbm.at[0, pl.ds(row0, _R), :], xbufs[0], lsems[0])
    for t in range(n_iter):
        c, b = divmod(t, _B)
        cur = t % 2
        if t + 1 < n_iter:
            nxt = (t + 1) % 2
            if t - 1 >= 0:
                stores[t - 1].wait()
            c2, b2 = divmod(t + 1, _B)
            loads[t + 1] = pltpu.async_copy(
                x_hbm.at[b2, pl.ds(row0 + c2 * _R, _R), :],
                xbufs[nxt], lsems[nxt])
        if b == 0 and c + 1 < _NCHUNK:
            pe_loads[c + 1] = pltpu.async_copy(
                pe_hbm.at[pl.ds(row0 + (c + 1) * _R, _R), :],
                pebufs[(c + 1) % 2], spe)
        if b == 0 and c > 0:
            pe_loads[c].wait()
        loads[t].wait()
        _fma_chunk(xbufs[cur], pebufs[c % 2])
        stores[t] = pltpu.async_copy(
            xbufs[cur], out_hbm.at[b, pl.ds(row0 + c * _R, _R), :],
            ssems[cur])
    stores[n_iter - 2].wait()
    stores[n_iter - 1].wait()


def _sc_call(x, pos_emb):
    mesh = plsc.VectorSubcoreMesh(core_axis_name="c", subcore_axis_name="s")
    run = functools.partial(
        pl.kernel,
        mesh=mesh,
        out_type=jax.ShapeDtypeStruct((_B, _SEQ, _D), jnp.float32),
        scratch_types=[
            pltpu.VMEM((_R, _D), jnp.float32),
            pltpu.VMEM((_R, _D), jnp.float32),
            pltpu.VMEM((_R, _D), jnp.float32),
            pltpu.VMEM((_R, _D), jnp.float32),
            pltpu.SemaphoreType.DMA,
            pltpu.SemaphoreType.DMA,
            pltpu.SemaphoreType.DMA,
            pltpu.SemaphoreType.DMA,
            pltpu.SemaphoreType.DMA,
        ],
    )(_sc_body)
    return run(x, pos_emb)


def kernel(x, pos_emb):
    b, s, d = x.shape
    return _sc_call(x, pos_emb[:s])
